# baseline (speedup 1.0000x reference)
import jax
import jax.numpy as jnp
from jax import lax
from jax.experimental import pallas as pl
from jax.experimental.pallas import tpu as pltpu
from jax.experimental.pallas import tpu_sc as plsc

B = 8
NN = 512
EN = 1024
K = 16
D = 768
V = 8192
MAXLEN = 2048
ACT = 1024
NR = B * NN
C = 16
NCH = 8
NV = D // 16


def _tc_body(lap_ref, w0t_ref, w1t_ref, ord_ref, p_ref, q3_ref):
    a = lap_ref[...]
    p = jnp.dot(a, w0t_ref[...], preferred_element_type=jnp.float32)
    q = jnp.dot(a, w1t_ref[...], preferred_element_type=jnp.float32)
    p_ref[...] = p
    q3_ref[0] = q + ord_ref[0:1, :]
    q3_ref[1] = q + ord_ref[1:2, :]


def _sc_body(nfid_h, efid_h, egu_h, egq_h, atom_h, edge_h, p_h, q3_h, out_h,
             idxa, idxe, idxu, idxq, bufs, zbuf,
             ga0, gb0, gc0, ga1, gb1, gc1, w0, w1, zs):
    cid = lax.axis_index("c")
    sid = lax.axis_index("s")
    wid = sid * 2 + cid
    b = wid // 4
    qq = wid % 4
    nbase = wid * (NCH * C)
    irow = wid * NCH
    nout = b * MAXLEN + qq * (NCH * C)
    eout = nout + NN
    zbase = b * MAXLEN + ACT + qq * (2 * NCH * C)

    gsems = [(ga0, gb0, gc0), (ga1, gb1, gc1)]
    wsems = [w0, w1]

    pltpu.sync_copy(nfid_h.at[pl.ds(irow, NCH)], idxa)
    pltpu.sync_copy(efid_h.at[pl.ds(irow, NCH)], idxe)
    pltpu.sync_copy(egu_h.at[pl.ds(irow, NCH)], idxu)
    pltpu.sync_copy(egq_h.at[pl.ds(irow, NCH)], idxq)

    def _zinit(_t, carry):
        for vv in range(NV):
            zbuf[_t, pl.ds(vv * 16, 16)] = jnp.zeros((16,), jnp.float32)
        return carry

    lax.fori_loop(0, C, _zinit, 0)

    NTOT = 2 * NCH

    def issue(i):
        s = i % 2
        b0, b1, b2 = bufs[s]
        sa, sb, sc = gsems[s]
        if i < NCH:
            ca = pltpu.async_copy(atom_h.at[idxa.at[i]], b0, sa)
            cp = pltpu.async_copy(p_h.at[pl.ds(nbase + i * C, C)], b1, sb)
            cq = pltpu.async_copy(q3_h.at[pl.ds(NR + nbase + i * C, C)], b2, sc)
        else:
            c = i - NCH
            ca = pltpu.async_copy(edge_h.at[idxe.at[c]], b0, sa)
            cp = pltpu.async_copy(p_h.at[idxu.at[c]], b1, sb)
            cq = pltpu.async_copy(q3_h.at[idxq.at[c]], b2, sc)
        return ca, cp, cq

    def outrow(i):
        return nout + i * C if i < NCH else eout + (i - NCH) * C

    gd = [None, None]
    wd = [None, None]
    zd = []

    gd[0] = issue(0)
    for i in range(NTOT):
        s = i % 2
        if i + 1 < NTOT:
            ns = (i + 1) % 2
            if wd[ns] is not None:
                wd[ns].wait()
                wd[ns] = None
            gd[ns] = issue(i + 1)
        for dsc in gd[s]:
            dsc.wait()
        b0, b1, b2 = bufs[s]

        def _accum(_t, carry):
            def _vv(j, carry2):
                for u in range(12):
                    sl = pl.ds(j * 192 + u * 16, 16)
                    plsc.addupdate(b0.at[_t, sl], b1[_t, sl] + b2[_t, sl])
                return carry2

            lax.fori_loop(0, NV // 12, _vv, 0)
            return carry

        lax.fori_loop(0, C, _accum, 0)
        wd[s] = pltpu.async_copy(b0, out_h.at[pl.ds(outrow(i), C)], wsems[s])
        zd.append(pltpu.async_copy(zbuf, out_h.at[pl.ds(zbase + i * C, C)], zs))

    for d in wd:
        d.wait()
    for d in zd:
        d.wait()


def kernel(node_data, node_num, lap_eigvec, edge_index, edge_data, edge_num,
           atom_emb, edge_emb, lap_W, order_emb):
    nfid = node_data.reshape(NR // C, C).astype(jnp.int32)
    efid = edge_data.reshape(B, EN)[:, :NN].reshape(NR // C, C).astype(jnp.int32)

    ei = edge_index.astype(jnp.int32)
    off = (jnp.arange(B, dtype=jnp.int32) * NN)[:, None]
    eu = ei[0].reshape(B, EN)[:, :NN] + off
    ev = ei[1].reshape(B, EN)[:, :NN] + off
    egq = (ev + jnp.where(eu == ev, NR, 0).astype(jnp.int32)).reshape(NR // C, C)
    egu = eu.reshape(NR // C, C)

    lapf = lap_eigvec.astype(jnp.float32)
    w0t = lap_W[:, :K].T
    w1t = lap_W[:, K:].T
    ordm = order_emb.astype(jnp.float32)

    P, Q3 = pl.pallas_call(
        _tc_body,
        grid=(16,),
        in_specs=[
            pl.BlockSpec((256, K), lambda i: (i, 0)),
            pl.BlockSpec((K, D), lambda i: (0, 0)),
            pl.BlockSpec((K, D), lambda i: (0, 0)),
            pl.BlockSpec((2, D), lambda i: (0, 0)),
        ],
        out_specs=[
            pl.BlockSpec((256, D), lambda i: (i, 0)),
            pl.BlockSpec((2, 256, D), lambda i: (0, i, 0)),
        ],
        out_shape=[
            jax.ShapeDtypeStruct((NR, D), jnp.float32),
            jax.ShapeDtypeStruct((2, NR, D), jnp.float32),
        ],
    )(lapf, w0t, w1t, ordm)
    Q3 = Q3.reshape(2 * NR, D)

    mesh = plsc.VectorSubcoreMesh(core_axis_name="c", subcore_axis_name="s")

    def body(nfid_h, efid_h, egu_h, egq_h, atom_h, edge_h, p_h, q3_h, out_h,
             idxa, idxe, idxu, idxq,
             b00, b01, b02, b10, b11, b12, zbuf,
             ga0, gb0, gc0, ga1, gb1, gc1, w0, w1, zs):
        _sc_body(nfid_h, efid_h, egu_h, egq_h, atom_h, edge_h, p_h, q3_h,
                 out_h, idxa, idxe, idxu, idxq,
                 [(b00, b01, b02), (b10, b11, b12)], zbuf,
                 ga0, gb0, gc0, ga1, gb1, gc1, w0, w1, zs)

    outflat = pl.kernel(
        body,
        out_type=jax.ShapeDtypeStruct((B * MAXLEN, D), jnp.float32),
        mesh=mesh,
        scratch_types=[
            pltpu.VMEM((NCH, C), jnp.int32),
            pltpu.VMEM((NCH, C), jnp.int32),
            pltpu.VMEM((NCH, C), jnp.int32),
            pltpu.VMEM((NCH, C), jnp.int32),
            pltpu.VMEM((C, D), jnp.float32),
            pltpu.VMEM((C, D), jnp.float32),
            pltpu.VMEM((C, D), jnp.float32),
            pltpu.VMEM((C, D), jnp.float32),
            pltpu.VMEM((C, D), jnp.float32),
            pltpu.VMEM((C, D), jnp.float32),
            pltpu.VMEM((C, D), jnp.float32),
            pltpu.SemaphoreType.DMA,
            pltpu.SemaphoreType.DMA,
            pltpu.SemaphoreType.DMA,
            pltpu.SemaphoreType.DMA,
            pltpu.SemaphoreType.DMA,
            pltpu.SemaphoreType.DMA,
            pltpu.SemaphoreType.DMA,
            pltpu.SemaphoreType.DMA,
            pltpu.SemaphoreType.DMA,
        ],
    )(nfid, efid, egu, egq, atom_emb, edge_emb, P, Q3)

    return outflat.reshape(B, MAXLEN, D)


# speedup vs baseline: 1.0642x; 1.0642x over previous
import jax
import jax.numpy as jnp
from jax import lax
from jax.experimental import pallas as pl
from jax.experimental.pallas import tpu as pltpu
from jax.experimental.pallas import tpu_sc as plsc

B = 8
NN = 512
EN = 1024
K = 16
D = 768
V = 8192
MAXLEN = 2048
ACT = 1024
NR = B * NN
C = 16
NCH = 8
NV = D // 16


def _tc_body(lap_ref, w0t_ref, w1t_ref, ord_ref, p_ref, q3_ref, r_ref):
    a = lap_ref[...]
    p = jnp.dot(a, w0t_ref[...], preferred_element_type=jnp.float32)
    q = jnp.dot(a, w1t_ref[...], preferred_element_type=jnp.float32)
    p_ref[...] = p
    q3_ref[0] = q + ord_ref[0:1, :]
    q3_ref[1] = q + ord_ref[1:2, :]
    r_ref[...] = p + q + ord_ref[1:2, :]


def _sc_body(nfid_h, efid_h, egu_h, egq_h, atom_h, edge_h, p_h, q3_h, r_h,
             out_h, idxa, idxe, idxu, idxq, bufs, zbuf,
             ga0, gb0, gc0, ga1, gb1, gc1, w0, w1, zs):
    cid = lax.axis_index("c")
    sid = lax.axis_index("s")
    wid = sid * 2 + cid
    b = wid // 4
    qq = wid % 4
    nbase = wid * (NCH * C)
    irow = wid * NCH
    nout = b * MAXLEN + qq * (NCH * C)
    eout = nout + NN
    zbase = b * MAXLEN + ACT + qq * (2 * NCH * C)

    gsems = [(ga0, gb0, gc0), (ga1, gb1, gc1)]
    wsems = [w0, w1]

    pltpu.sync_copy(nfid_h.at[pl.ds(irow, NCH)], idxa)
    pltpu.sync_copy(efid_h.at[pl.ds(irow, NCH)], idxe)
    pltpu.sync_copy(egu_h.at[pl.ds(irow, NCH)], idxu)
    pltpu.sync_copy(egq_h.at[pl.ds(irow, NCH)], idxq)

    def _zinit(_t, carry):
        for vv in range(NV):
            zbuf[_t, pl.ds(vv * 16, 16)] = jnp.zeros((16,), jnp.float32)
        return carry

    lax.fori_loop(0, C, _zinit, 0)

    NTOT = 2 * NCH

    def issue(i):
        s = i % 2
        b0, b1, b2 = bufs[s]
        sa, sb, sc = gsems[s]
        if i < NCH:
            ca = pltpu.async_copy(atom_h.at[idxa.at[i]], b0, sa)
            cp = pltpu.async_copy(r_h.at[pl.ds(nbase + i * C, C)], b1, sb)
            return (ca, cp)
        c = i - NCH
        ca = pltpu.async_copy(edge_h.at[idxe.at[c]], b0, sa)
        cp = pltpu.async_copy(p_h.at[idxu.at[c]], b1, sb)
        cq = pltpu.async_copy(q3_h.at[idxq.at[c]], b2, sc)
        return (ca, cp, cq)

    def outrow(i):
        return nout + i * C if i < NCH else eout + (i - NCH) * C

    gd = [None, None]
    wd = [None, None]
    zd = []

    gd[0] = issue(0)
    for i in range(NTOT):
        s = i % 2
        if i + 1 < NTOT:
            ns = (i + 1) % 2
            if wd[ns] is not None:
                wd[ns].wait()
                wd[ns] = None
            gd[ns] = issue(i + 1)
        for dsc in gd[s]:
            dsc.wait()
        b0, b1, b2 = bufs[s]

        if i < NCH:
            def _accum2(_t, carry):
                def _vv(j, carry2):
                    for u in range(12):
                        sl = pl.ds(j * 192 + u * 16, 16)
                        plsc.addupdate(b0.at[_t, sl], b1[_t, sl])
                    return carry2

                lax.fori_loop(0, NV // 12, _vv, 0)
                return carry

            lax.fori_loop(0, C, _accum2, 0)
        else:
            def _accum3(_t, carry):
                def _vv(j, carry2):
                    for u in range(12):
                        sl = pl.ds(j * 192 + u * 16, 16)
                        plsc.addupdate(b0.at[_t, sl], b1[_t, sl] + b2[_t, sl])
                    return carry2

                lax.fori_loop(0, NV // 12, _vv, 0)
                return carry

            lax.fori_loop(0, C, _accum3, 0)
        wd[s] = pltpu.async_copy(b0, out_h.at[pl.ds(outrow(i), C)], wsems[s])
        zd.append(pltpu.async_copy(zbuf, out_h.at[pl.ds(zbase + i * C, C)], zs))

    for d in wd:
        d.wait()
    for d in zd:
        d.wait()


def kernel(node_data, node_num, lap_eigvec, edge_index, edge_data, edge_num,
           atom_emb, edge_emb, lap_W, order_emb):
    nfid = node_data.reshape(NR // C, C).astype(jnp.int32)
    efid = edge_data.reshape(B, EN)[:, :NN].reshape(NR // C, C).astype(jnp.int32)

    ei = edge_index.astype(jnp.int32)
    off = (jnp.arange(B, dtype=jnp.int32) * NN)[:, None]
    eu = ei[0].reshape(B, EN)[:, :NN] + off
    ev = ei[1].reshape(B, EN)[:, :NN] + off
    egq = (ev + jnp.where(eu == ev, NR, 0).astype(jnp.int32)).reshape(NR // C, C)
    egu = eu.reshape(NR // C, C)

    lapf = lap_eigvec.astype(jnp.float32)
    w0t = lap_W[:, :K].T
    w1t = lap_W[:, K:].T
    ordm = order_emb.astype(jnp.float32)

    P, Q3, R = pl.pallas_call(
        _tc_body,
        grid=(16,),
        in_specs=[
            pl.BlockSpec((256, K), lambda i: (i, 0)),
            pl.BlockSpec((K, D), lambda i: (0, 0)),
            pl.BlockSpec((K, D), lambda i: (0, 0)),
            pl.BlockSpec((2, D), lambda i: (0, 0)),
        ],
        out_specs=[
            pl.BlockSpec((256, D), lambda i: (i, 0)),
            pl.BlockSpec((2, 256, D), lambda i: (0, i, 0)),
            pl.BlockSpec((256, D), lambda i: (i, 0)),
        ],
        out_shape=[
            jax.ShapeDtypeStruct((NR, D), jnp.float32),
            jax.ShapeDtypeStruct((2, NR, D), jnp.float32),
            jax.ShapeDtypeStruct((NR, D), jnp.float32),
        ],
    )(lapf, w0t, w1t, ordm)
    Q3 = Q3.reshape(2 * NR, D)

    mesh = plsc.VectorSubcoreMesh(core_axis_name="c", subcore_axis_name="s")

    def body(nfid_h, efid_h, egu_h, egq_h, atom_h, edge_h, p_h, q3_h, r_h,
             out_h, idxa, idxe, idxu, idxq,
             b00, b01, b02, b10, b11, b12, zbuf,
             ga0, gb0, gc0, ga1, gb1, gc1, w0, w1, zs):
        _sc_body(nfid_h, efid_h, egu_h, egq_h, atom_h, edge_h, p_h, q3_h, r_h,
                 out_h, idxa, idxe, idxu, idxq,
                 [(b00, b01, b02), (b10, b11, b12)], zbuf,
                 ga0, gb0, gc0, ga1, gb1, gc1, w0, w1, zs)

    outflat = pl.kernel(
        body,
        out_type=jax.ShapeDtypeStruct((B * MAXLEN, D), jnp.float32),
        mesh=mesh,
        scratch_types=[
            pltpu.VMEM((NCH, C), jnp.int32),
            pltpu.VMEM((NCH, C), jnp.int32),
            pltpu.VMEM((NCH, C), jnp.int32),
            pltpu.VMEM((NCH, C), jnp.int32),
            pltpu.VMEM((C, D), jnp.float32),
            pltpu.VMEM((C, D), jnp.float32),
            pltpu.VMEM((C, D), jnp.float32),
            pltpu.VMEM((C, D), jnp.float32),
            pltpu.VMEM((C, D), jnp.float32),
            pltpu.VMEM((C, D), jnp.float32),
            pltpu.VMEM((C, D), jnp.float32),
            pltpu.SemaphoreType.DMA,
            pltpu.SemaphoreType.DMA,
            pltpu.SemaphoreType.DMA,
            pltpu.SemaphoreType.DMA,
            pltpu.SemaphoreType.DMA,
            pltpu.SemaphoreType.DMA,
            pltpu.SemaphoreType.DMA,
            pltpu.SemaphoreType.DMA,
            pltpu.SemaphoreType.DMA,
        ],
    )(nfid, efid, egu, egq, atom_emb, edge_emb, P, Q3, R)

    return outflat.reshape(B, MAXLEN, D)


# speedup vs baseline: 1.2033x; 1.1307x over previous
import jax
import jax.numpy as jnp
from jax import lax
from jax.experimental import pallas as pl
from jax.experimental.pallas import tpu as pltpu
from jax.experimental.pallas import tpu_sc as plsc

B = 8
NN = 512
EN = 1024
K = 16
D = 768
V = 8192
MAXLEN = 2048
ACT = 1024
NR = B * NN
C = 16
NCH = 8
TPW = NCH * C


def _sc_body(nfid_h, efid_h, egu_h, egv_h, atom_h, edge_h, lap_h,
             featn_h, feate_h, lapu_h, lapv_h,
             idxa, idxe, idxu, idxv, b00, b10, lbu, lbv,
             ga0, ga1, w0, w1, lsu, lsv):
    cid = lax.axis_index("c")
    sid = lax.axis_index("s")
    wid = sid * 2 + cid
    rbase = wid * TPW
    irow = wid * NCH

    pltpu.sync_copy(nfid_h.at[pl.ds(irow, NCH)], idxa)
    pltpu.sync_copy(efid_h.at[pl.ds(irow, NCH)], idxe)
    pltpu.sync_copy(egu_h.at[wid], idxu)
    pltpu.sync_copy(egv_h.at[wid], idxv)

    cu = pltpu.async_copy(lap_h.at[idxu], lbu, lsu)
    cv = pltpu.async_copy(lap_h.at[idxv], lbv, lsv)

    bufs = [b00, b10]
    gsem = [ga0, ga1]
    wsem = [w0, w1]

    def issue(i):
        s = i % 2
        if i < NCH:
            return pltpu.async_copy(atom_h.at[idxa.at[i]], bufs[s], gsem[s])
        return pltpu.async_copy(edge_h.at[idxe.at[i - NCH]], bufs[s], gsem[s])

    def wrow(i):
        if i < NCH:
            return featn_h.at[pl.ds(rbase + i * C, C)]
        return feate_h.at[pl.ds(rbase + (i - NCH) * C, C)]

    gd = [None, None]
    wd = [None, None]
    gd[0] = issue(0)
    for i in range(2 * NCH):
        s = i % 2
        if i + 1 < 2 * NCH:
            ns = (i + 1) % 2
            if wd[ns] is not None:
                wd[ns].wait()
                wd[ns] = None
            gd[ns] = issue(i + 1)
        gd[s].wait()
        wd[s] = pltpu.async_copy(bufs[s], wrow(i), wsem[s])
    for d in wd:
        if d is not None:
            d.wait()

    cu.wait()
    cv.wait()
    su = pltpu.async_copy(lbu, lapu_h.at[pl.ds(rbase, TPW)], lsu)
    sv = pltpu.async_copy(lbv, lapv_h.at[pl.ds(rbase, TPW)], lsv)
    su.wait()
    sv.wait()


def _tc_body(lapn_ref, lapu_ref, lapv_ref, featn_ref, feate_ref, eqf_ref,
             w0t_ref, w1t_ref, ord_ref, out_ref):
    j = pl.program_id(1)

    @pl.when(j >= 4)
    def _():
        out_ref[...] = jnp.zeros((1, 256, D), jnp.float32)

    @pl.when(j < 4)
    def _():
        is_node = j < 2
        u = jnp.where(is_node, lapn_ref[...], lapu_ref[:, :K])
        v = jnp.where(is_node, lapn_ref[...], lapv_ref[:, :K])
        f = jnp.where(is_node, featn_ref[...], feate_ref[...])
        eqf = jnp.where(is_node, jnp.ones((256, 1), jnp.float32),
                        eqf_ref[:, 0:1])
        le = (jnp.dot(u, w0t_ref[...], preferred_element_type=jnp.float32)
              + jnp.dot(v, w1t_ref[...], preferred_element_type=jnp.float32))
        o0 = ord_ref[0:1, :]
        oe = o0 + eqf * (ord_ref[1:2, :] - o0)
        out_ref[0] = f + le + oe


def kernel(node_data, node_num, lap_eigvec, edge_index, edge_data, edge_num,
           atom_emb, edge_emb, lap_W, order_emb):
    nfid = node_data.reshape(NR // C, C).astype(jnp.int32)
    efid = edge_data.reshape(B, EN)[:, :NN].reshape(NR // C, C).astype(jnp.int32)

    ei = edge_index.astype(jnp.int32)
    off = (jnp.arange(B, dtype=jnp.int32) * NN)[:, None]
    eu = ei[0].reshape(B, EN)[:, :NN] + off
    ev = ei[1].reshape(B, EN)[:, :NN] + off
    eqf = jnp.broadcast_to(
        (eu == ev).astype(jnp.float32).reshape(NR, 1), (NR, 8))
    egu = eu.reshape(NR // TPW, TPW)
    egv = ev.reshape(NR // TPW, TPW)

    lapf = lap_eigvec.astype(jnp.float32)
    lap128 = jnp.pad(lapf, ((0, 0), (0, 128 - K)))
    w0t = lap_W[:, :K].T
    w1t = lap_W[:, K:].T
    ordm = order_emb.astype(jnp.float32)

    mesh = plsc.VectorSubcoreMesh(core_axis_name="c", subcore_axis_name="s")
    featn, feate, lapu, lapv = pl.kernel(
        _sc_body,
        out_type=[
            jax.ShapeDtypeStruct((NR, D), jnp.float32),
            jax.ShapeDtypeStruct((NR, D), jnp.float32),
            jax.ShapeDtypeStruct((NR, 128), jnp.float32),
            jax.ShapeDtypeStruct((NR, 128), jnp.float32),
        ],
        mesh=mesh,
        scratch_types=[
            pltpu.VMEM((NCH, C), jnp.int32),
            pltpu.VMEM((NCH, C), jnp.int32),
            pltpu.VMEM((TPW,), jnp.int32),
            pltpu.VMEM((TPW,), jnp.int32),
            pltpu.VMEM((C, D), jnp.float32),
            pltpu.VMEM((C, D), jnp.float32),
            pltpu.VMEM((TPW, 128), jnp.float32),
            pltpu.VMEM((TPW, 128), jnp.float32),
            pltpu.SemaphoreType.DMA,
            pltpu.SemaphoreType.DMA,
            pltpu.SemaphoreType.DMA,
            pltpu.SemaphoreType.DMA,
            pltpu.SemaphoreType.DMA,
            pltpu.SemaphoreType.DMA,
        ],
    )(nfid, efid, egu, egv, atom_emb, edge_emb, lap128)

    def nmap(b, j):
        return (b * 2 + jnp.minimum(j, 1), 0)

    def emap(b, j):
        return (b * 2 + jnp.clip(j - 2, 0, 1), 0)

    out = pl.pallas_call(
        _tc_body,
        grid=(B, MAXLEN // 256),
        in_specs=[
            pl.BlockSpec((256, K), nmap),
            pl.BlockSpec((256, 128), emap),
            pl.BlockSpec((256, 128), emap),
            pl.BlockSpec((256, D), nmap),
            pl.BlockSpec((256, D), emap),
            pl.BlockSpec((256, 8), emap),
            pl.BlockSpec((K, D), lambda b, j: (0, 0)),
            pl.BlockSpec((K, D), lambda b, j: (0, 0)),
            pl.BlockSpec((2, D), lambda b, j: (0, 0)),
        ],
        out_specs=pl.BlockSpec((1, 256, D), lambda b, j: (b, j, 0)),
        out_shape=jax.ShapeDtypeStruct((B, MAXLEN, D), jnp.float32),
    )(lapf, lapu, lapv, featn, feate, eqf, w0t, w1t, ordm)

    return out


# speedup vs baseline: 1.3950x; 1.1593x over previous
import jax
import jax.numpy as jnp
from jax import lax
from jax.experimental import pallas as pl
from jax.experimental.pallas import tpu as pltpu
from jax.experimental.pallas import tpu_sc as plsc

B = 8
NN = 512
EN = 1024
K = 16
D = 768
V = 8192
MAXLEN = 2048
ACT = 1024
NR = B * NN
C = 16
NCH = 8
TPW = NCH * C


def _sc_body(nfid_h, efid_h, egu_h, egv_h, atom_h, edge_h, lap_h,
             feat_h, lapu_h, lapv_h,
             idxa, idxe, idxu, idxv, b00, b10, lbu, lbv,
             ga0, ga1, w0, w1, lsu, lsv):
    cid = lax.axis_index("c")
    sid = lax.axis_index("s")
    wid = sid * 2 + cid
    b = wid // 4
    qq = wid % 4
    rbase = wid * TPW
    nb = b * ACT + qq * TPW
    eb = nb + NN
    irow = wid * NCH

    pltpu.sync_copy(nfid_h.at[pl.ds(irow, NCH)], idxa)
    pltpu.sync_copy(efid_h.at[pl.ds(irow, NCH)], idxe)
    pltpu.sync_copy(egu_h.at[wid], idxu)
    pltpu.sync_copy(egv_h.at[wid], idxv)

    cu = pltpu.async_copy(lap_h.at[idxu], lbu, lsu)
    cv = pltpu.async_copy(lap_h.at[idxv], lbv, lsv)

    bufs = [b00, b10]
    gsem = [ga0, ga1]
    wsem = [w0, w1]

    def issue(i):
        s = i % 2
        if i < NCH:
            return pltpu.async_copy(atom_h.at[idxa.at[i]], bufs[s], gsem[s])
        return pltpu.async_copy(edge_h.at[idxe.at[i - NCH]], bufs[s], gsem[s])

    def wrow(i):
        if i < NCH:
            return feat_h.at[pl.ds(nb + i * C, C)]
        return feat_h.at[pl.ds(eb + (i - NCH) * C, C)]

    gd = [None, None]
    wd = [None, None]
    gd[0] = issue(0)
    for i in range(2 * NCH):
        s = i % 2
        if i + 1 < 2 * NCH:
            ns = (i + 1) % 2
            if wd[ns] is not None:
                wd[ns].wait()
                wd[ns] = None
            gd[ns] = issue(i + 1)
        gd[s].wait()
        wd[s] = pltpu.async_copy(bufs[s], wrow(i), wsem[s])
    for d in wd:
        if d is not None:
            d.wait()

    cu.wait()
    cv.wait()
    su = pltpu.async_copy(lbu, lapu_h.at[pl.ds(rbase, TPW)], lsu)
    sv = pltpu.async_copy(lbv, lapv_h.at[pl.ds(rbase, TPW)], lsv)
    su.wait()
    sv.wait()


def _tc_body(lapn_ref, lapu_ref, lapv_ref, feat_ref, eqf_ref,
             w0t_ref, w1t_ref, ord_ref, out_ref):
    j = pl.program_id(1)

    @pl.when(j == 1)
    def _():
        out_ref[...] = jnp.zeros((1, ACT, D), jnp.float32)

    @pl.when(j == 0)
    def _():
        lapn = lapn_ref[...]
        u = jnp.concatenate([lapn, lapu_ref[:, :K]], axis=0)
        v = jnp.concatenate([lapn, lapv_ref[:, :K]], axis=0)
        eqf = jnp.concatenate(
            [jnp.ones((NN, 1), jnp.float32), eqf_ref[:, 0:1]], axis=0)
        le = (jnp.dot(u, w0t_ref[...], preferred_element_type=jnp.float32)
              + jnp.dot(v, w1t_ref[...], preferred_element_type=jnp.float32))
        o0 = ord_ref[0:1, :]
        oe = o0 + eqf * (ord_ref[1:2, :] - o0)
        out_ref[0] = feat_ref[...] + le + oe


def kernel(node_data, node_num, lap_eigvec, edge_index, edge_data, edge_num,
           atom_emb, edge_emb, lap_W, order_emb):
    nfid = node_data.reshape(NR // C, C).astype(jnp.int32)
    efid = edge_data.reshape(B, EN)[:, :NN].reshape(NR // C, C).astype(jnp.int32)

    ei = edge_index.astype(jnp.int32)
    off = (jnp.arange(B, dtype=jnp.int32) * NN)[:, None]
    eu = ei[0].reshape(B, EN)[:, :NN] + off
    ev = ei[1].reshape(B, EN)[:, :NN] + off
    eqf = jnp.broadcast_to(
        (eu == ev).astype(jnp.float32).reshape(NR, 1), (NR, 8))
    egu = eu.reshape(NR // TPW, TPW)
    egv = ev.reshape(NR // TPW, TPW)

    lapf = lap_eigvec.astype(jnp.float32)
    lap128 = jnp.pad(lapf, ((0, 0), (0, 128 - K)))
    w0t = lap_W[:, :K].T
    w1t = lap_W[:, K:].T
    ordm = order_emb.astype(jnp.float32)

    mesh = plsc.VectorSubcoreMesh(core_axis_name="c", subcore_axis_name="s")
    feat, lapu, lapv = pl.kernel(
        _sc_body,
        out_type=[
            jax.ShapeDtypeStruct((B * ACT, D), jnp.float32),
            jax.ShapeDtypeStruct((NR, 128), jnp.float32),
            jax.ShapeDtypeStruct((NR, 128), jnp.float32),
        ],
        mesh=mesh,
        scratch_types=[
            pltpu.VMEM((NCH, C), jnp.int32),
            pltpu.VMEM((NCH, C), jnp.int32),
            pltpu.VMEM((TPW,), jnp.int32),
            pltpu.VMEM((TPW,), jnp.int32),
            pltpu.VMEM((C, D), jnp.float32),
            pltpu.VMEM((C, D), jnp.float32),
            pltpu.VMEM((TPW, 128), jnp.float32),
            pltpu.VMEM((TPW, 128), jnp.float32),
            pltpu.SemaphoreType.DMA,
            pltpu.SemaphoreType.DMA,
            pltpu.SemaphoreType.DMA,
            pltpu.SemaphoreType.DMA,
            pltpu.SemaphoreType.DMA,
            pltpu.SemaphoreType.DMA,
        ],
    )(nfid, efid, egu, egv, atom_emb, edge_emb, lap128)

    out = pl.pallas_call(
        _tc_body,
        grid=(B, 2),
        in_specs=[
            pl.BlockSpec((NN, K), lambda b, j: (b, 0)),
            pl.BlockSpec((NN, 128), lambda b, j: (b, 0)),
            pl.BlockSpec((NN, 128), lambda b, j: (b, 0)),
            pl.BlockSpec((ACT, D), lambda b, j: (b, 0)),
            pl.BlockSpec((NN, 8), lambda b, j: (b, 0)),
            pl.BlockSpec((K, D), lambda b, j: (0, 0)),
            pl.BlockSpec((K, D), lambda b, j: (0, 0)),
            pl.BlockSpec((2, D), lambda b, j: (0, 0)),
        ],
        out_specs=pl.BlockSpec((1, ACT, D), lambda b, j: (b, j, 0)),
        out_shape=jax.ShapeDtypeStruct((B, MAXLEN, D), jnp.float32),
    )(lapf, lapu, lapv, feat, eqf, w0t, w1t, ordm)

    return out


# speedup vs baseline: 1.4450x; 1.0359x over previous
import jax
import jax.numpy as jnp
from jax import lax
from jax.experimental import pallas as pl
from jax.experimental.pallas import tpu as pltpu
from jax.experimental.pallas import tpu_sc as plsc

B = 8
NN = 512
EN = 1024
K = 16
D = 768
V = 8192
MAXLEN = 2048
ACT = 1024
NR = B * NN
C = 16
NCH = 8
TPW = NCH * C


def _sc_body(nfid_h, efid_h, atom_h, edge_h, feat_h,
             idxa, idxe, b00, b10, ga0, ga1, w0, w1):
    cid = lax.axis_index("c")
    sid = lax.axis_index("s")
    wid = sid * 2 + cid
    b = wid // 4
    qq = wid % 4
    nb = b * ACT + qq * TPW
    eb = nb + NN
    irow = wid * NCH

    pltpu.sync_copy(nfid_h.at[pl.ds(irow, NCH)], idxa)
    pltpu.sync_copy(efid_h.at[pl.ds(irow, NCH)], idxe)

    bufs = [b00, b10]
    gsem = [ga0, ga1]
    wsem = [w0, w1]

    def issue(i):
        s = i % 2
        if i < NCH:
            return pltpu.async_copy(atom_h.at[idxa.at[i]], bufs[s], gsem[s])
        return pltpu.async_copy(edge_h.at[idxe.at[i - NCH]], bufs[s], gsem[s])

    def wrow(i):
        if i < NCH:
            return feat_h.at[pl.ds(nb + i * C, C)]
        return feat_h.at[pl.ds(eb + (i - NCH) * C, C)]

    gd = [None, None]
    wd = [None, None]
    gd[0] = issue(0)
    for i in range(2 * NCH):
        s = i % 2
        if i + 1 < 2 * NCH:
            ns = (i + 1) % 2
            if wd[ns] is not None:
                wd[ns].wait()
                wd[ns] = None
            gd[ns] = issue(i + 1)
        gd[s].wait()
        wd[s] = pltpu.async_copy(bufs[s], wrow(i), wsem[s])
    for d in wd:
        if d is not None:
            d.wait()


def _tc_body(lapn_ref, eu_ref, ev_ref, feat_ref,
             w0t_ref, w1t_ref, ord_ref, out_ref):
    j = pl.program_id(1)

    @pl.when(j == 1)
    def _():
        out_ref[...] = jnp.zeros((1, ACT, D), jnp.float32)

    @pl.when(j == 0)
    def _():
        lapn = lapn_ref[...]
        uu = eu_ref[:, 0:1]
        vv = ev_ref[:, 0:1]
        col = lax.broadcasted_iota(jnp.int32, (NN, NN), 1)
        e0 = jnp.dot((uu == col).astype(jnp.float32), lapn,
                     preferred_element_type=jnp.float32)
        e1 = jnp.dot((vv == col).astype(jnp.float32), lapn,
                     preferred_element_type=jnp.float32)
        u = jnp.concatenate([lapn, e0], axis=0)
        v = jnp.concatenate([lapn, e1], axis=0)
        eqf = jnp.concatenate(
            [jnp.ones((NN, 1), jnp.float32),
             (uu == vv).astype(jnp.float32)], axis=0)
        le = (jnp.dot(u, w0t_ref[...], preferred_element_type=jnp.float32)
              + jnp.dot(v, w1t_ref[...], preferred_element_type=jnp.float32))
        o0 = ord_ref[0:1, :]
        oe = o0 + eqf * (ord_ref[1:2, :] - o0)
        out_ref[0] = feat_ref[...] + le + oe


def kernel(node_data, node_num, lap_eigvec, edge_index, edge_data, edge_num,
           atom_emb, edge_emb, lap_W, order_emb):
    nfid = node_data.reshape(NR // C, C).astype(jnp.int32)
    efid = edge_data.reshape(B, EN)[:, :NN].reshape(NR // C, C).astype(jnp.int32)

    ei = edge_index.astype(jnp.int32)
    eu8 = jnp.broadcast_to(
        ei[0].reshape(B, EN)[:, :NN].reshape(NR, 1), (NR, 8))
    ev8 = jnp.broadcast_to(
        ei[1].reshape(B, EN)[:, :NN].reshape(NR, 1), (NR, 8))

    lapf = lap_eigvec.astype(jnp.float32)
    w0t = lap_W[:, :K].T
    w1t = lap_W[:, K:].T
    ordm = order_emb.astype(jnp.float32)

    mesh = plsc.VectorSubcoreMesh(core_axis_name="c", subcore_axis_name="s")
    feat = pl.kernel(
        _sc_body,
        out_type=jax.ShapeDtypeStruct((B * ACT, D), jnp.float32),
        mesh=mesh,
        scratch_types=[
            pltpu.VMEM((NCH, C), jnp.int32),
            pltpu.VMEM((NCH, C), jnp.int32),
            pltpu.VMEM((C, D), jnp.float32),
            pltpu.VMEM((C, D), jnp.float32),
            pltpu.SemaphoreType.DMA,
            pltpu.SemaphoreType.DMA,
            pltpu.SemaphoreType.DMA,
            pltpu.SemaphoreType.DMA,
        ],
    )(nfid, efid, atom_emb, edge_emb)

    out = pl.pallas_call(
        _tc_body,
        grid=(B, 2),
        in_specs=[
            pl.BlockSpec((NN, K), lambda b, j: (b, 0)),
            pl.BlockSpec((NN, 8), lambda b, j: (b, 0)),
            pl.BlockSpec((NN, 8), lambda b, j: (b, 0)),
            pl.BlockSpec((ACT, D), lambda b, j: (b, 0)),
            pl.BlockSpec((K, D), lambda b, j: (0, 0)),
            pl.BlockSpec((K, D), lambda b, j: (0, 0)),
            pl.BlockSpec((2, D), lambda b, j: (0, 0)),
        ],
        out_specs=pl.BlockSpec((1, ACT, D), lambda b, j: (b, j, 0)),
        out_shape=jax.ShapeDtypeStruct((B, MAXLEN, D), jnp.float32),
    )(lapf, eu8, ev8, feat, w0t, w1t, ordm)

    return out


# speedup vs baseline: 1.7774x; 1.2300x over previous
import jax
import jax.numpy as jnp
from jax import lax
from jax.experimental import pallas as pl
from jax.experimental.pallas import tpu as pltpu
from jax.experimental.pallas import tpu_sc as plsc

B = 8
NN = 512
EN = 1024
K = 16
D = 768
V = 8192
MAXLEN = 2048
ACT = 1024
NR = B * NN
C = 16
NCH = 8
TPW = NCH * C


def _sc_body(nfid_h, efid_h, atom_h, edge_h, feat_h,
             idxa, idxe, b00, b10, ga0, ga1, w0, w1):
    cid = lax.axis_index("c")
    sid = lax.axis_index("s")
    wid = sid * 2 + cid
    b = wid // 4
    qq = wid % 4
    nb = b * ACT + qq * TPW
    eb = nb + NN
    irow = wid * NCH

    pltpu.sync_copy(nfid_h.at[pl.ds(irow, NCH)], idxa)
    pltpu.sync_copy(efid_h.at[pl.ds(irow, NCH)], idxe)

    bufs = [b00, b10]
    gsem = [ga0, ga1]
    wsem = [w0, w1]

    def issue(i):
        s = i % 2
        if i < NCH:
            return pltpu.async_copy(atom_h.at[idxa.at[i]], bufs[s], gsem[s])
        return pltpu.async_copy(edge_h.at[idxe.at[i - NCH]], bufs[s], gsem[s])

    def wrow(i):
        if i < NCH:
            return feat_h.at[pl.ds(nb + i * C, C)]
        return feat_h.at[pl.ds(eb + (i - NCH) * C, C)]

    gd = [None, None]
    wd = [None, None]
    gd[0] = issue(0)
    for i in range(2 * NCH):
        s = i % 2
        if i + 1 < 2 * NCH:
            ns = (i + 1) % 2
            if wd[ns] is not None:
                wd[ns].wait()
                wd[ns] = None
            gd[ns] = issue(i + 1)
        gd[s].wait()
        wd[s] = pltpu.async_copy(bufs[s], wrow(i), wsem[s])
    for d in wd:
        if d is not None:
            d.wait()


def _tc_zero_body(out_ref):
    out_ref[...] = jnp.zeros((1, ACT, D), jnp.float32)


def _tc_body(lapn_ref, eu_ref, ev_ref, feat_ref,
             w0t_ref, w1t_ref, ord_ref, zo_ref, out_ref):
    del zo_ref
    if True:
        lapn = lapn_ref[...]
        uu = eu_ref[:, 0:1]
        vv = ev_ref[:, 0:1]
        col = lax.broadcasted_iota(jnp.int32, (NN, NN), 1)
        e0 = jnp.dot((uu == col).astype(jnp.float32), lapn,
                     preferred_element_type=jnp.float32)
        e1 = jnp.dot((vv == col).astype(jnp.float32), lapn,
                     preferred_element_type=jnp.float32)
        u = jnp.concatenate([lapn, e0], axis=0)
        v = jnp.concatenate([lapn, e1], axis=0)
        eqf = jnp.concatenate(
            [jnp.ones((NN, 1), jnp.float32),
             (uu == vv).astype(jnp.float32)], axis=0)
        le = (jnp.dot(u, w0t_ref[...], preferred_element_type=jnp.float32)
              + jnp.dot(v, w1t_ref[...], preferred_element_type=jnp.float32))
        o0 = ord_ref[0:1, :]
        oe = o0 + eqf * (ord_ref[1:2, :] - o0)
        out_ref[0] = feat_ref[...] + le + oe


def kernel(node_data, node_num, lap_eigvec, edge_index, edge_data, edge_num,
           atom_emb, edge_emb, lap_W, order_emb):
    nfid = node_data.reshape(NR // C, C).astype(jnp.int32)
    efid = edge_data.reshape(B, EN)[:, :NN].reshape(NR // C, C).astype(jnp.int32)

    ei = edge_index.astype(jnp.int32)
    eu8 = jnp.broadcast_to(
        ei[0].reshape(B, EN)[:, :NN].reshape(NR, 1), (NR, 8))
    ev8 = jnp.broadcast_to(
        ei[1].reshape(B, EN)[:, :NN].reshape(NR, 1), (NR, 8))

    lapf = lap_eigvec.astype(jnp.float32)
    w0t = lap_W[:, :K].T
    w1t = lap_W[:, K:].T
    ordm = order_emb.astype(jnp.float32)

    mesh = plsc.VectorSubcoreMesh(core_axis_name="c", subcore_axis_name="s")
    feat = pl.kernel(
        _sc_body,
        out_type=jax.ShapeDtypeStruct((B * ACT, D), jnp.float32),
        mesh=mesh,
        scratch_types=[
            pltpu.VMEM((NCH, C), jnp.int32),
            pltpu.VMEM((NCH, C), jnp.int32),
            pltpu.VMEM((C, D), jnp.float32),
            pltpu.VMEM((C, D), jnp.float32),
            pltpu.SemaphoreType.DMA,
            pltpu.SemaphoreType.DMA,
            pltpu.SemaphoreType.DMA,
            pltpu.SemaphoreType.DMA,
        ],
    )(nfid, efid, atom_emb, edge_emb)

    zhalf = pl.pallas_call(
        _tc_zero_body,
        grid=(B,),
        out_specs=pl.BlockSpec((1, ACT, D), lambda b: (b, 1, 0)),
        out_shape=jax.ShapeDtypeStruct((B, MAXLEN, D), jnp.float32),
    )()

    out = pl.pallas_call(
        _tc_body,
        grid=(B,),
        in_specs=[
            pl.BlockSpec((NN, K), lambda b: (b, 0)),
            pl.BlockSpec((NN, 8), lambda b: (b, 0)),
            pl.BlockSpec((NN, 8), lambda b: (b, 0)),
            pl.BlockSpec((ACT, D), lambda b: (b, 0)),
            pl.BlockSpec((K, D), lambda b: (0, 0)),
            pl.BlockSpec((K, D), lambda b: (0, 0)),
            pl.BlockSpec((2, D), lambda b: (0, 0)),
            pl.BlockSpec(memory_space=pl.ANY),
        ],
        out_specs=pl.BlockSpec((1, ACT, D), lambda b: (b, 0, 0)),
        out_shape=jax.ShapeDtypeStruct((B, MAXLEN, D), jnp.float32),
        input_output_aliases={7: 0},
    )(lapf, eu8, ev8, feat, w0t, w1t, ordm, zhalf)

    return out


# speedup vs baseline: 1.7903x; 1.0073x over previous
import jax
import jax.numpy as jnp
from jax import lax
from jax.experimental import pallas as pl
from jax.experimental.pallas import tpu as pltpu
from jax.experimental.pallas import tpu_sc as plsc

B = 8
NN = 512
EN = 1024
K = 16
D = 768
V = 8192
MAXLEN = 2048
ACT = 1024
NR = B * NN
C = 32
NCH = 4
TPW = NCH * C


def _sc_body(nfid_h, efid_h, atom_h, edge_h, feat_h,
             idxa, idxe, b00, b10, ga0, ga1, w0, w1):
    cid = lax.axis_index("c")
    sid = lax.axis_index("s")
    wid = sid * 2 + cid
    b = wid // 4
    qq = wid % 4
    nb = b * ACT + qq * TPW
    eb = nb + NN
    irow = wid * NCH

    pltpu.sync_copy(nfid_h.at[pl.ds(irow, NCH)], idxa)
    pltpu.sync_copy(efid_h.at[pl.ds(irow, NCH)], idxe)

    bufs = [b00, b10]
    gsem = [ga0, ga1]
    wsem = [w0, w1]

    def issue(i):
        s = i % 2
        if i < NCH:
            return pltpu.async_copy(atom_h.at[idxa.at[i]], bufs[s], gsem[s])
        return pltpu.async_copy(edge_h.at[idxe.at[i - NCH]], bufs[s], gsem[s])

    def wrow(i):
        if i < NCH:
            return feat_h.at[pl.ds(nb + i * C, C)]
        return feat_h.at[pl.ds(eb + (i - NCH) * C, C)]

    gd = [None, None]
    wd = [None, None]
    gd[0] = issue(0)
    for i in range(2 * NCH):
        s = i % 2
        if i + 1 < 2 * NCH:
            ns = (i + 1) % 2
            if wd[ns] is not None:
                wd[ns].wait()
                wd[ns] = None
            gd[ns] = issue(i + 1)
        gd[s].wait()
        wd[s] = pltpu.async_copy(bufs[s], wrow(i), wsem[s])
    for d in wd:
        if d is not None:
            d.wait()


def _tc_zero_body(out_ref):
    out_ref[...] = jnp.zeros((1, ACT, D), jnp.float32)


def _tc_body(lapn_ref, eu_ref, ev_ref, feat_ref,
             w0t_ref, w1t_ref, ord_ref, zo_ref, out_ref):
    del zo_ref
    if True:
        lapn = lapn_ref[...]
        uu = eu_ref[:, 0:1]
        vv = ev_ref[:, 0:1]
        col = lax.broadcasted_iota(jnp.int32, (NN, NN), 1)
        e0 = jnp.dot((uu == col).astype(jnp.float32), lapn,
                     preferred_element_type=jnp.float32)
        e1 = jnp.dot((vv == col).astype(jnp.float32), lapn,
                     preferred_element_type=jnp.float32)
        u = jnp.concatenate([lapn, e0], axis=0)
        v = jnp.concatenate([lapn, e1], axis=0)
        eqf = jnp.concatenate(
            [jnp.ones((NN, 1), jnp.float32),
             (uu == vv).astype(jnp.float32)], axis=0)
        le = (jnp.dot(u, w0t_ref[...], preferred_element_type=jnp.float32)
              + jnp.dot(v, w1t_ref[...], preferred_element_type=jnp.float32))
        o0 = ord_ref[0:1, :]
        oe = o0 + eqf * (ord_ref[1:2, :] - o0)
        out_ref[0] = feat_ref[...] + le + oe


def kernel(node_data, node_num, lap_eigvec, edge_index, edge_data, edge_num,
           atom_emb, edge_emb, lap_W, order_emb):
    nfid = node_data.reshape(NR // C, C).astype(jnp.int32)
    efid = edge_data.reshape(B, EN)[:, :NN].reshape(NR // C, C).astype(jnp.int32)

    ei = edge_index.astype(jnp.int32)
    eu8 = jnp.broadcast_to(
        ei[0].reshape(B, EN)[:, :NN].reshape(NR, 1), (NR, 8))
    ev8 = jnp.broadcast_to(
        ei[1].reshape(B, EN)[:, :NN].reshape(NR, 1), (NR, 8))

    lapf = lap_eigvec.astype(jnp.float32)
    w0t = lap_W[:, :K].T
    w1t = lap_W[:, K:].T
    ordm = order_emb.astype(jnp.float32)

    mesh = plsc.VectorSubcoreMesh(core_axis_name="c", subcore_axis_name="s")
    feat = pl.kernel(
        _sc_body,
        out_type=jax.ShapeDtypeStruct((B * ACT, D), jnp.float32),
        mesh=mesh,
        scratch_types=[
            pltpu.VMEM((NCH, C), jnp.int32),
            pltpu.VMEM((NCH, C), jnp.int32),
            pltpu.VMEM((C, D), jnp.float32),
            pltpu.VMEM((C, D), jnp.float32),
            pltpu.SemaphoreType.DMA,
            pltpu.SemaphoreType.DMA,
            pltpu.SemaphoreType.DMA,
            pltpu.SemaphoreType.DMA,
        ],
    )(nfid, efid, atom_emb, edge_emb)

    zhalf = pl.pallas_call(
        _tc_zero_body,
        grid=(B,),
        out_specs=pl.BlockSpec((1, ACT, D), lambda b: (b, 1, 0)),
        out_shape=jax.ShapeDtypeStruct((B, MAXLEN, D), jnp.float32),
    )()

    out = pl.pallas_call(
        _tc_body,
        grid=(B,),
        in_specs=[
            pl.BlockSpec((NN, K), lambda b: (b, 0)),
            pl.BlockSpec((NN, 8), lambda b: (b, 0)),
            pl.BlockSpec((NN, 8), lambda b: (b, 0)),
            pl.BlockSpec((ACT, D), lambda b: (b, 0)),
            pl.BlockSpec((K, D), lambda b: (0, 0)),
            pl.BlockSpec((K, D), lambda b: (0, 0)),
            pl.BlockSpec((2, D), lambda b: (0, 0)),
            pl.BlockSpec(memory_space=pl.ANY),
        ],
        out_specs=pl.BlockSpec((1, ACT, D), lambda b: (b, 0, 0)),
        out_shape=jax.ShapeDtypeStruct((B, MAXLEN, D), jnp.float32),
        input_output_aliases={7: 0},
    )(lapf, eu8, ev8, feat, w0t, w1t, ordm, zhalf)

    return out


# speedup vs baseline: 1.8145x; 1.0135x over previous
import jax
import jax.numpy as jnp
from jax import lax
from jax.experimental import pallas as pl
from jax.experimental.pallas import tpu as pltpu
from jax.experimental.pallas import tpu_sc as plsc

B = 8
NN = 512
EN = 1024
K = 16
D = 768
V = 8192
MAXLEN = 2048
ACT = 1024
NR = B * NN
C = 32
NCH = 4
TPW = NCH * C


def _sc_body(nfid_h, efid_h, atom_h, edge_h, feat_h,
             idxa, idxe, b00, b10, b20, b30,
             ga0, ga1, ga2, ga3, w0, w1, w2, w3):
    cid = lax.axis_index("c")
    sid = lax.axis_index("s")
    wid = sid * 2 + cid
    b = wid // 4
    qq = wid % 4
    nb = b * ACT + qq * TPW
    eb = nb + NN
    irow = wid * NCH

    pltpu.sync_copy(nfid_h.at[pl.ds(irow, NCH)], idxa)
    pltpu.sync_copy(efid_h.at[pl.ds(irow, NCH)], idxe)

    bufs = [b00, b10, b20, b30]
    gsem = [ga0, ga1, ga2, ga3]
    wsem = [w0, w1, w2, w3]
    NB = 4

    def issue(i):
        s = i % NB
        if i < NCH:
            return pltpu.async_copy(atom_h.at[idxa.at[i]], bufs[s], gsem[s])
        return pltpu.async_copy(edge_h.at[idxe.at[i - NCH]], bufs[s], gsem[s])

    def wrow(i):
        if i < NCH:
            return feat_h.at[pl.ds(nb + i * C, C)]
        return feat_h.at[pl.ds(eb + (i - NCH) * C, C)]

    gd = [None] * NB
    wd = [None] * NB
    for p in range(NB - 1):
        gd[p] = issue(p)
    for i in range(2 * NCH):
        s = i % NB
        nxt = i + NB - 1
        if nxt < 2 * NCH:
            ns = nxt % NB
            if wd[ns] is not None:
                wd[ns].wait()
                wd[ns] = None
            gd[ns] = issue(nxt)
        gd[s].wait()
        wd[s] = pltpu.async_copy(bufs[s], wrow(i), wsem[s])
    for d in wd:
        if d is not None:
            d.wait()


def _tc_zero_body(out_ref):
    out_ref[...] = jnp.zeros((1, ACT, D), jnp.float32)


def _tc_body(lapn_ref, eu_ref, ev_ref, feat_ref,
             w0t_ref, w1t_ref, ord_ref, zo_ref, out_ref):
    del zo_ref
    if True:
        lapn = lapn_ref[...]
        uu = eu_ref[:, 0:1]
        vv = ev_ref[:, 0:1]
        col = lax.broadcasted_iota(jnp.int32, (NN, NN), 1)
        e0 = jnp.dot((uu == col).astype(jnp.float32), lapn,
                     preferred_element_type=jnp.float32)
        e1 = jnp.dot((vv == col).astype(jnp.float32), lapn,
                     preferred_element_type=jnp.float32)
        u = jnp.concatenate([lapn, e0], axis=0)
        v = jnp.concatenate([lapn, e1], axis=0)
        eqf = jnp.concatenate(
            [jnp.ones((NN, 1), jnp.float32),
             (uu == vv).astype(jnp.float32)], axis=0)
        le = (jnp.dot(u, w0t_ref[...], preferred_element_type=jnp.float32)
              + jnp.dot(v, w1t_ref[...], preferred_element_type=jnp.float32))
        o0 = ord_ref[0:1, :]
        oe = o0 + eqf * (ord_ref[1:2, :] - o0)
        out_ref[0] = feat_ref[...] + le + oe


def kernel(node_data, node_num, lap_eigvec, edge_index, edge_data, edge_num,
           atom_emb, edge_emb, lap_W, order_emb):
    nfid = node_data.reshape(NR // C, C).astype(jnp.int32)
    efid = edge_data.reshape(B, EN)[:, :NN].reshape(NR // C, C).astype(jnp.int32)

    ei = edge_index.astype(jnp.int32)
    eu8 = jnp.broadcast_to(
        ei[0].reshape(B, EN)[:, :NN].reshape(NR, 1), (NR, 8))
    ev8 = jnp.broadcast_to(
        ei[1].reshape(B, EN)[:, :NN].reshape(NR, 1), (NR, 8))

    lapf = lap_eigvec.astype(jnp.float32)
    w0t = lap_W[:, :K].T
    w1t = lap_W[:, K:].T
    ordm = order_emb.astype(jnp.float32)

    mesh = plsc.VectorSubcoreMesh(core_axis_name="c", subcore_axis_name="s")
    feat = pl.kernel(
        _sc_body,
        out_type=jax.ShapeDtypeStruct((B * ACT, D), jnp.float32),
        mesh=mesh,
        scratch_types=[
            pltpu.VMEM((NCH, C), jnp.int32),
            pltpu.VMEM((NCH, C), jnp.int32),
            pltpu.VMEM((C, D), jnp.float32),
            pltpu.VMEM((C, D), jnp.float32),
            pltpu.VMEM((C, D), jnp.float32),
            pltpu.VMEM((C, D), jnp.float32),
            pltpu.SemaphoreType.DMA,
            pltpu.SemaphoreType.DMA,
            pltpu.SemaphoreType.DMA,
            pltpu.SemaphoreType.DMA,
            pltpu.SemaphoreType.DMA,
            pltpu.SemaphoreType.DMA,
            pltpu.SemaphoreType.DMA,
            pltpu.SemaphoreType.DMA,
        ],
    )(nfid, efid, atom_emb, edge_emb)

    zhalf = pl.pallas_call(
        _tc_zero_body,
        grid=(B,),
        out_specs=pl.BlockSpec((1, ACT, D), lambda b: (b, 1, 0)),
        out_shape=jax.ShapeDtypeStruct((B, MAXLEN, D), jnp.float32),
    )()

    out = pl.pallas_call(
        _tc_body,
        grid=(B,),
        in_specs=[
            pl.BlockSpec((NN, K), lambda b: (b, 0)),
            pl.BlockSpec((NN, 8), lambda b: (b, 0)),
            pl.BlockSpec((NN, 8), lambda b: (b, 0)),
            pl.BlockSpec((ACT, D), lambda b: (b, 0)),
            pl.BlockSpec((K, D), lambda b: (0, 0)),
            pl.BlockSpec((K, D), lambda b: (0, 0)),
            pl.BlockSpec((2, D), lambda b: (0, 0)),
            pl.BlockSpec(memory_space=pl.ANY),
        ],
        out_specs=pl.BlockSpec((1, ACT, D), lambda b: (b, 0, 0)),
        out_shape=jax.ShapeDtypeStruct((B, MAXLEN, D), jnp.float32),
        input_output_aliases={7: 0},
    )(lapf, eu8, ev8, feat, w0t, w1t, ordm, zhalf)

    return out


# speedup vs baseline: 1.8457x; 1.0172x over previous
import jax
import jax.numpy as jnp
from jax import lax
from jax.experimental import pallas as pl
from jax.experimental.pallas import tpu as pltpu
from jax.experimental.pallas import tpu_sc as plsc

B = 8
NN = 512
EN = 1024
K = 16
D = 768
V = 8192
MAXLEN = 2048
ACT = 1024
NR = B * NN
HB = 4
C = 32
NCH = 2
TPW = NCH * C
NBUF = 4


def _sc_body(nfid_h, efid_h, atom_h, edge_h, feat_h,
             idxa, idxe, b00, b10, b20, b30,
             ga0, ga1, ga2, ga3, w0, w1, w2, w3):
    cid = lax.axis_index("c")
    sid = lax.axis_index("s")
    wid = sid * 2 + cid
    b = wid // 8
    qq = wid % 8
    nb = b * ACT + qq * TPW
    eb = nb + NN
    irow = wid * NCH

    pltpu.sync_copy(nfid_h.at[pl.ds(irow, NCH)], idxa)
    pltpu.sync_copy(efid_h.at[pl.ds(irow, NCH)], idxe)

    bufs = [b00, b10, b20, b30]
    gsem = [ga0, ga1, ga2, ga3]
    wsem = [w0, w1, w2, w3]

    def issue(i):
        s = i % NBUF
        if i < NCH:
            return pltpu.async_copy(atom_h.at[idxa.at[i]], bufs[s], gsem[s])
        return pltpu.async_copy(edge_h.at[idxe.at[i - NCH]], bufs[s], gsem[s])

    def wrow(i):
        if i < NCH:
            return feat_h.at[pl.ds(nb + i * C, C)]
        return feat_h.at[pl.ds(eb + (i - NCH) * C, C)]

    NTOT = 2 * NCH
    gd = [None] * NBUF
    wd = [None] * NBUF
    for p in range(min(NBUF - 1, NTOT)):
        gd[p] = issue(p)
    for i in range(NTOT):
        s = i % NBUF
        nxt = i + NBUF - 1
        if nxt < NTOT:
            ns = nxt % NBUF
            if wd[ns] is not None:
                wd[ns].wait()
                wd[ns] = None
            gd[ns] = issue(nxt)
        gd[s].wait()
        wd[s] = pltpu.async_copy(bufs[s], wrow(i), wsem[s])
    for d in wd:
        if d is not None:
            d.wait()


def _tc_zero_body(out_ref):
    out_ref[...] = jnp.zeros((1, ACT, D), jnp.float32)


def _tc_body(lapn_ref, eu_ref, ev_ref, feat_ref,
             w0t_ref, w1t_ref, ord_ref, zo_ref, out_ref):
    del zo_ref
    lapn = lapn_ref[...]
    uu = eu_ref[:, 0:1]
    vv = ev_ref[:, 0:1]
    col = lax.broadcasted_iota(jnp.int32, (NN, NN), 1)
    e0 = jnp.dot((uu == col).astype(jnp.float32), lapn,
                 preferred_element_type=jnp.float32)
    e1 = jnp.dot((vv == col).astype(jnp.float32), lapn,
                 preferred_element_type=jnp.float32)
    u = jnp.concatenate([lapn, e0], axis=0)
    v = jnp.concatenate([lapn, e1], axis=0)
    eqf = jnp.concatenate(
        [jnp.ones((NN, 1), jnp.float32),
         (uu == vv).astype(jnp.float32)], axis=0)
    le = (jnp.dot(u, w0t_ref[...], preferred_element_type=jnp.float32)
          + jnp.dot(v, w1t_ref[...], preferred_element_type=jnp.float32))
    o0 = ord_ref[0:1, :]
    oe = o0 + eqf * (ord_ref[1:2, :] - o0)
    out_ref[0] = feat_ref[...] + le + oe


def _sc_gather(nfid, efid, atom_emb, edge_emb):
    mesh = plsc.VectorSubcoreMesh(core_axis_name="c", subcore_axis_name="s")
    return pl.kernel(
        _sc_body,
        out_type=jax.ShapeDtypeStruct((HB * ACT, D), jnp.float32),
        mesh=mesh,
        scratch_types=[
            pltpu.VMEM((NCH, C), jnp.int32),
            pltpu.VMEM((NCH, C), jnp.int32),
            pltpu.VMEM((C, D), jnp.float32),
            pltpu.VMEM((C, D), jnp.float32),
            pltpu.VMEM((C, D), jnp.float32),
            pltpu.VMEM((C, D), jnp.float32),
            pltpu.SemaphoreType.DMA,
            pltpu.SemaphoreType.DMA,
            pltpu.SemaphoreType.DMA,
            pltpu.SemaphoreType.DMA,
            pltpu.SemaphoreType.DMA,
            pltpu.SemaphoreType.DMA,
            pltpu.SemaphoreType.DMA,
            pltpu.SemaphoreType.DMA,
        ],
    )(nfid, efid, atom_emb, edge_emb)


def _tc_combine(half, lapf, eu8, ev8, feat, w0t, w1t, ordm, prev):
    off = half * HB
    return pl.pallas_call(
        _tc_body,
        grid=(HB,),
        in_specs=[
            pl.BlockSpec((NN, K), lambda b: (b + off, 0)),
            pl.BlockSpec((NN, 8), lambda b: (b + off, 0)),
            pl.BlockSpec((NN, 8), lambda b: (b + off, 0)),
            pl.BlockSpec((ACT, D), lambda b: (b, 0)),
            pl.BlockSpec((K, D), lambda b: (0, 0)),
            pl.BlockSpec((K, D), lambda b: (0, 0)),
            pl.BlockSpec((2, D), lambda b: (0, 0)),
            pl.BlockSpec(memory_space=pl.ANY),
        ],
        out_specs=pl.BlockSpec((1, ACT, D), lambda b: (b + off, 0, 0)),
        out_shape=jax.ShapeDtypeStruct((B, MAXLEN, D), jnp.float32),
        input_output_aliases={7: 0},
    )(lapf, eu8, ev8, feat, w0t, w1t, ordm, prev)


def kernel(node_data, node_num, lap_eigvec, edge_index, edge_data, edge_num,
           atom_emb, edge_emb, lap_W, order_emb):
    nfid = node_data.reshape(B * NN // C, C).astype(jnp.int32)
    efid = edge_data.reshape(B, EN)[:, :NN].reshape(B * NN // C, C).astype(jnp.int32)
    HROWS = HB * NN // C

    ei = edge_index.astype(jnp.int32)
    eu8 = jnp.broadcast_to(
        ei[0].reshape(B, EN)[:, :NN].reshape(NR, 1), (NR, 8))
    ev8 = jnp.broadcast_to(
        ei[1].reshape(B, EN)[:, :NN].reshape(NR, 1), (NR, 8))

    lapf = lap_eigvec.astype(jnp.float32)
    w0t = lap_W[:, :K].T
    w1t = lap_W[:, K:].T
    ordm = order_emb.astype(jnp.float32)

    feat_a = _sc_gather(nfid[:HROWS], efid[:HROWS], atom_emb, edge_emb)
    feat_b = _sc_gather(nfid[HROWS:], efid[HROWS:], atom_emb, edge_emb)

    zhalf = pl.pallas_call(
        _tc_zero_body,
        grid=(B,),
        out_specs=pl.BlockSpec((1, ACT, D), lambda b: (b, 1, 0)),
        out_shape=jax.ShapeDtypeStruct((B, MAXLEN, D), jnp.float32),
    )()

    out = _tc_combine(0, lapf, eu8, ev8, feat_a, w0t, w1t, ordm, zhalf)
    out = _tc_combine(1, lapf, eu8, ev8, feat_b, w0t, w1t, ordm, out)
    return out


# speedup vs baseline: 1.8504x; 1.0025x over previous
import jax
import jax.numpy as jnp
from jax import lax
from jax.experimental import pallas as pl
from jax.experimental.pallas import tpu as pltpu
from jax.experimental.pallas import tpu_sc as plsc

B = 8
NN = 512
EN = 1024
K = 16
D = 768
V = 8192
MAXLEN = 2048
ACT = 1024
NR = B * NN
HB = 4
C = 64
NCH = 1
TPW = NCH * C
NBUF = 2


def _sc_body(nfid_h, efid_h, atom_h, edge_h, feat_h,
             idxa, idxe, b00, b10,
             ga0, ga1, w0, w1):
    cid = lax.axis_index("c")
    sid = lax.axis_index("s")
    wid = sid * 2 + cid
    b = wid // 8
    qq = wid % 8
    nb = b * ACT + qq * TPW
    eb = nb + NN
    irow = wid * NCH

    pltpu.sync_copy(nfid_h.at[pl.ds(irow, NCH)], idxa)
    pltpu.sync_copy(efid_h.at[pl.ds(irow, NCH)], idxe)

    bufs = [b00, b10]
    gsem = [ga0, ga1]
    wsem = [w0, w1]

    def issue(i):
        s = i % NBUF
        if i < NCH:
            return pltpu.async_copy(atom_h.at[idxa.at[i]], bufs[s], gsem[s])
        return pltpu.async_copy(edge_h.at[idxe.at[i - NCH]], bufs[s], gsem[s])

    def wrow(i):
        if i < NCH:
            return feat_h.at[pl.ds(nb + i * C, C)]
        return feat_h.at[pl.ds(eb + (i - NCH) * C, C)]

    NTOT = 2 * NCH
    gd = [None] * NBUF
    wd = [None] * NBUF
    for p in range(min(NBUF - 1, NTOT)):
        gd[p] = issue(p)
    for i in range(NTOT):
        s = i % NBUF
        nxt = i + NBUF - 1
        if nxt < NTOT:
            ns = nxt % NBUF
            if wd[ns] is not None:
                wd[ns].wait()
                wd[ns] = None
            gd[ns] = issue(nxt)
        gd[s].wait()
        wd[s] = pltpu.async_copy(bufs[s], wrow(i), wsem[s])
    for d in wd:
        if d is not None:
            d.wait()


def _tc_zero_body(out_ref):
    out_ref[...] = jnp.zeros((1, ACT, D), jnp.float32)


def _tc_body(lapn_ref, eu_ref, ev_ref, feat_ref,
             w0t_ref, w1t_ref, ord_ref, zo_ref, out_ref):
    del zo_ref
    lapn = lapn_ref[...]
    uu = eu_ref[:, 0:1]
    vv = ev_ref[:, 0:1]
    col = lax.broadcasted_iota(jnp.int32, (NN, NN), 1)
    e0 = jnp.dot((uu == col).astype(jnp.float32), lapn,
                 preferred_element_type=jnp.float32)
    e1 = jnp.dot((vv == col).astype(jnp.float32), lapn,
                 preferred_element_type=jnp.float32)
    u = jnp.concatenate([lapn, e0], axis=0)
    v = jnp.concatenate([lapn, e1], axis=0)
    eqf = jnp.concatenate(
        [jnp.ones((NN, 1), jnp.float32),
         (uu == vv).astype(jnp.float32)], axis=0)
    le = (jnp.dot(u, w0t_ref[...], preferred_element_type=jnp.float32)
          + jnp.dot(v, w1t_ref[...], preferred_element_type=jnp.float32))
    o0 = ord_ref[0:1, :]
    oe = o0 + eqf * (ord_ref[1:2, :] - o0)
    out_ref[0] = feat_ref[...] + le + oe


def _sc_gather(nfid, efid, atom_emb, edge_emb):
    mesh = plsc.VectorSubcoreMesh(core_axis_name="c", subcore_axis_name="s")
    return pl.kernel(
        _sc_body,
        out_type=jax.ShapeDtypeStruct((HB * ACT, D), jnp.float32),
        mesh=mesh,
        scratch_types=[
            pltpu.VMEM((NCH, C), jnp.int32),
            pltpu.VMEM((NCH, C), jnp.int32),
            pltpu.VMEM((C, D), jnp.float32),
            pltpu.VMEM((C, D), jnp.float32),
            pltpu.SemaphoreType.DMA,
            pltpu.SemaphoreType.DMA,
            pltpu.SemaphoreType.DMA,
            pltpu.SemaphoreType.DMA,
        ],
    )(nfid, efid, atom_emb, edge_emb)


def _tc_combine(half, lapf, eu8, ev8, feat, w0t, w1t, ordm, prev):
    off = half * HB
    return pl.pallas_call(
        _tc_body,
        grid=(HB,),
        in_specs=[
            pl.BlockSpec((NN, K), lambda b: (b + off, 0)),
            pl.BlockSpec((NN, 8), lambda b: (b + off, 0)),
            pl.BlockSpec((NN, 8), lambda b: (b + off, 0)),
            pl.BlockSpec((ACT, D), lambda b: (b, 0)),
            pl.BlockSpec((K, D), lambda b: (0, 0)),
            pl.BlockSpec((K, D), lambda b: (0, 0)),
            pl.BlockSpec((2, D), lambda b: (0, 0)),
            pl.BlockSpec(memory_space=pl.ANY),
        ],
        out_specs=pl.BlockSpec((1, ACT, D), lambda b: (b + off, 0, 0)),
        out_shape=jax.ShapeDtypeStruct((B, MAXLEN, D), jnp.float32),
        input_output_aliases={7: 0},
    )(lapf, eu8, ev8, feat, w0t, w1t, ordm, prev)


def kernel(node_data, node_num, lap_eigvec, edge_index, edge_data, edge_num,
           atom_emb, edge_emb, lap_W, order_emb):
    nfid = node_data.reshape(B * NN // C, C).astype(jnp.int32)
    efid = edge_data.reshape(B, EN)[:, :NN].reshape(B * NN // C, C).astype(jnp.int32)
    HROWS = HB * NN // C

    ei = edge_index.astype(jnp.int32)
    eu8 = jnp.broadcast_to(
        ei[0].reshape(B, EN)[:, :NN].reshape(NR, 1), (NR, 8))
    ev8 = jnp.broadcast_to(
        ei[1].reshape(B, EN)[:, :NN].reshape(NR, 1), (NR, 8))

    lapf = lap_eigvec.astype(jnp.float32)
    w0t = lap_W[:, :K].T
    w1t = lap_W[:, K:].T
    ordm = order_emb.astype(jnp.float32)

    feat_a = _sc_gather(nfid[:HROWS], efid[:HROWS], atom_emb, edge_emb)
    feat_b = _sc_gather(nfid[HROWS:], efid[HROWS:], atom_emb, edge_emb)

    zhalf = pl.pallas_call(
        _tc_zero_body,
        grid=(B,),
        out_specs=pl.BlockSpec((1, ACT, D), lambda b: (b, 1, 0)),
        out_shape=jax.ShapeDtypeStruct((B, MAXLEN, D), jnp.float32),
    )()

    out = _tc_combine(0, lapf, eu8, ev8, feat_a, w0t, w1t, ordm, zhalf)
    out = _tc_combine(1, lapf, eu8, ev8, feat_b, w0t, w1t, ordm, out)
    return out
